# baseline (speedup 1.0000x reference)
import functools

import jax
import jax.numpy as jnp
from jax import lax
from jax.experimental import pallas as pl
from jax.experimental.pallas import tpu as pltpu
from jax.experimental.pallas import tpu_sc as plsc

BATCH = 16384
DIM = 64
NCOL = 7813
_NCOLP = 7888

_INFO = plsc.get_sparse_core_info()
_NC = _INFO.num_cores
_NS = _INFO.num_subcores
_NW = _NC * _NS
_L = 16
_K = 8
_FB = 64
_NG = BATCH // _L

_mesh = plsc.VectorSubcoreMesh(core_axis_name="c", subcore_axis_name="s")


@functools.partial(
    pl.kernel,
    mesh=_mesh,
    out_type=jax.ShapeDtypeStruct((BATCH, 2 * DIM), jnp.float32),
    scratch_types=[
        pltpu.VMEM((BATCH + _L,), jnp.int32),
        pltpu.VMEM((BATCH + 2 * _L,), jnp.int32),
        pltpu.VMEM((_NCOLP,), jnp.int32),
        pltpu.VMEM((_NCOLP,), jnp.int32),
        pltpu.VMEM((NCOL // _NW + 4 * _L,), jnp.int32),
        [pltpu.VMEM((DIM, 128), jnp.float32) for _ in range(_K)],
        pltpu.VMEM((_FB, 2 * DIM), jnp.float32),
        pltpu.VMEM((_FB,), jnp.int32),
        [pltpu.SemaphoreType.DMA for _ in range(_K)],
    ],
    compiler_params=pltpu.CompilerParams(needs_layout_passes=False),
)
def _gather_concat(ids_hbm, tt_hbm, out_hbm, ids_v, spos_v, hist_v, offs_v,
                   used_v, cols, rows_v, rpos_v, gsems):
    wid = lax.axis_index("s") * _NC + lax.axis_index("c")

    pltpu.sync_copy(ids_hbm, ids_v.at[pl.ds(0, BATCH)])

    zvi = jnp.zeros((_L,), jnp.int32)
    zvf = jnp.zeros((_L,), jnp.float32)
    iota = lax.iota(jnp.int32, _L)
    iotas = [iota + q * _L for q in range(DIM // _L)]
    m0 = iota == 0

    def _hz(g, carry):
        hist_v[pl.ds(g * _L, _L)] = zvi
        return carry

    lax.fori_loop(0, _NCOLP // _L, _hz, 0)

    def _hist(g, carry):
        j = ids_v[pl.ds(g * _L, _L)] >> 7
        cnt, last = plsc.scan_count(j)
        plsc.addupdate_scatter(hist_v, [j], cnt, mask=last)
        return carry

    lax.fori_loop(0, _NG, _hist, 0)

    def _scan(g, carry):
        c = hist_v[pl.ds(g * _L, _L)]
        s = plsc.cumsum(c)
        offs_v[pl.ds(g * _L, _L)] = carry + s - c
        return carry + s[_L - 1]

    lax.fori_loop(0, _NCOLP // _L, _scan, 0)

    def _place(g, carry):
        j = ids_v[pl.ds(g * _L, _L)] >> 7
        cnt, last = plsc.scan_count(j)
        old = plsc.load_gather(offs_v, [j])
        plsc.store_scatter(spos_v, [old + cnt - 1], iota + g * _L)
        plsc.addupdate_scatter(offs_v, [j], cnt, mask=last)
        return carry

    lax.fori_loop(0, _NG, _place, 0)

    lo = (wid * NCOL) >> 5
    hi = ((wid + 1) * NCOL) >> 5
    span = hi - lo

    def _compact(g, ucnt):
        cid = lo + g * _L + iota
        h = hist_v[pl.ds(lo + g * _L, _L)]
        m = (h > 0) & (cid < hi)
        plsc.store_compressed(used_v.at[pl.ds(ucnt, _L)], cid, mask=m)
        return ucnt + plsc.all_reduce_population_count(m)[0]

    ucnt = lax.fori_loop(0, (span + _L - 1) >> 4, _compact, 0)

    def _zrow(i, carry):
        for q in range(DIM // _L):
            rows_v[i, pl.ds(DIM + q * _L, _L)] = zvf
        return carry

    lax.fori_loop(0, _FB, _zrow, 0)

    cvec0 = used_v[pl.ds(0, _L)]
    for k in range(_K):
        @pl.when(k < ucnt)
        def _(k=k):
            coloff = pl.multiple_of(cvec0[k] * 128, 128)
            pltpu.make_async_copy(
                tt_hbm.at[:, pl.ds(coloff, 128)], cols[k], gsems[k]
            ).start()

    def _sweep(o, rc):
        cvec = used_v[pl.ds(o * _K, 2 * _K)]
        for k in range(_K):
            uu = o * _K + k
            live = uu < ucnt

            @pl.when(live)
            def _(k=k):
                pltpu.make_async_copy(
                    tt_hbm.at[:, pl.ds(0, 128)], cols[k], gsems[k]
                ).wait()

            c = jnp.where(live, cvec[k], 0)
            n_c = jnp.where(live, hist_v[pl.ds(c, _L)][0], 0)
            s_c = offs_v[pl.ds(c, _L)][0] - n_c

            def _chunk(t, rc, k=k, n_c=n_c, s_c=s_c):
                rem = n_c - t * _L
                pv = spos_v[pl.ds(s_c + t * _L, _L)]
                lanes = plsc.load_gather(ids_v, [pv], mask=iota < rem) & 127
                for k2 in range(_L):
                    row = (rc + k2) & (_FB - 1)

                    @pl.when(k2 < rem)
                    def _(k2=k2, row=row, pv=pv, lanes=lanes):
                        lanev = jnp.full((_L,), lanes[k2], dtype=jnp.int32)
                        for q in range(DIM // _L):
                            v = plsc.load_gather(cols[k], [iotas[q], lanev])
                            rows_v[row, pl.ds(q * _L, _L)] = v
                        plsc.store_scatter(
                            rpos_v,
                            [jnp.full((_L,), row, dtype=jnp.int32)],
                            jnp.full((_L,), pv[k2], dtype=jnp.int32),
                            mask=m0,
                        )

                    @pl.when((k2 < rem) & (row == _FB - 1))
                    def _():
                        pltpu.sync_copy(rows_v, out_hbm.at[rpos_v])

                return rc + jnp.clip(rem, 0, _L)

            rc = lax.fori_loop(0, (n_c + _L - 1) >> 4, _chunk, rc)

            nu = uu + _K

            @pl.when(nu < ucnt)
            def _(k=k):
                coloff = pl.multiple_of(cvec[k + _K] * 128, 128)
                pltpu.make_async_copy(
                    tt_hbm.at[:, pl.ds(coloff, 128)], cols[k], gsems[k]
                ).start()

        return rc

    rc = lax.fori_loop(0, (ucnt + _K - 1) >> 3, _sweep, 0)

    tail = rc & (_FB - 1)

    @pl.when(tail > 0)
    def _():
        lastp = plsc.load_gather(
            rpos_v, [jnp.full((_L,), tail - 1, dtype=jnp.int32)]
        )

        def _pad(t, carry):
            i = tail + t
            for q in range(2 * DIM // _L):
                rows_v[i, pl.ds(q * _L, _L)] = rows_v[tail - 1, pl.ds(q * _L, _L)]
            return carry

        lax.fori_loop(0, _FB - tail, _pad, 0)

        for t in range(_FB // _L):
            idxv = tail + t * _L + iota
            plsc.store_scatter(rpos_v, [idxv], lastp, mask=idxv < _FB)
        pltpu.sync_copy(rows_v, out_hbm.at[rpos_v])


def kernel(entity_ids, entity_table):
    ids = entity_ids.astype(jnp.int32)
    return _gather_concat(ids, entity_table.T)


# speedup vs baseline: 1.0037x; 1.0037x over previous
import functools

import jax
import jax.numpy as jnp
from jax import lax
from jax.experimental import pallas as pl
from jax.experimental.pallas import tpu as pltpu
from jax.experimental.pallas import tpu_sc as plsc

BATCH = 16384
DIM = 64
NCOL = 7813
_NCOLP = 7888

_INFO = plsc.get_sparse_core_info()
_NC = _INFO.num_cores
_NS = _INFO.num_subcores
_NW = _NC * _NS
_L = 16
_K = 8
_FB = 64
_NG = BATCH // _L

_mesh = plsc.VectorSubcoreMesh(core_axis_name="c", subcore_axis_name="s")


@functools.partial(
    pl.kernel,
    mesh=_mesh,
    out_type=jax.ShapeDtypeStruct((BATCH, 2 * DIM), jnp.float32),
    scratch_types=[
        pltpu.VMEM((BATCH + _L,), jnp.int32),
        pltpu.VMEM((BATCH + 2 * _L,), jnp.int32),
        pltpu.VMEM((_NCOLP,), jnp.int32),
        pltpu.VMEM((_NCOLP,), jnp.int32),
        pltpu.VMEM((NCOL // _NW + 4 * _L,), jnp.int32),
        [pltpu.VMEM((DIM, 128), jnp.float32) for _ in range(_K)],
        pltpu.VMEM((_FB, 2 * DIM), jnp.float32),
        pltpu.VMEM((_FB,), jnp.int32),
        [pltpu.SemaphoreType.DMA for _ in range(_K)],
    ],
    compiler_params=pltpu.CompilerParams(needs_layout_passes=False),
)
def _gather_concat(ids_hbm, tt_hbm, out_hbm, ids_v, spos_v, hist_v, offs_v,
                   used_v, cols, rows_v, rpos_v, gsems):
    wid = lax.axis_index("s") * _NC + lax.axis_index("c")

    pltpu.sync_copy(ids_hbm, ids_v.at[pl.ds(0, BATCH)])

    zvi = jnp.zeros((_L,), jnp.int32)
    zvf = jnp.zeros((_L,), jnp.float32)
    iota = lax.iota(jnp.int32, _L)
    iotas = [iota + q * _L for q in range(DIM // _L)]
    m0 = iota == 0

    def _hz(g, carry):
        hist_v[pl.ds(g * _L, _L)] = zvi
        return carry

    lax.fori_loop(0, _NCOLP // _L, _hz, 0)

    def _hist(g, carry):
        for u in range(4):
            j = ids_v[pl.ds((g * 4 + u) * _L, _L)] >> 7
            cnt, last = plsc.scan_count(j)
            plsc.addupdate_scatter(hist_v, [j], cnt, mask=last)
        return carry

    lax.fori_loop(0, _NG // 4, _hist, 0)

    def _scan(g, carry):
        c = hist_v[pl.ds(g * _L, _L)]
        s = plsc.cumsum(c)
        offs_v[pl.ds(g * _L, _L)] = carry + s - c
        return carry + s[_L - 1]

    lax.fori_loop(0, _NCOLP // _L, _scan, 0)

    def _place(g, carry):
        for u in range(4):
            gg = g * 4 + u
            j = ids_v[pl.ds(gg * _L, _L)] >> 7
            cnt, last = plsc.scan_count(j)
            old = plsc.load_gather(offs_v, [j])
            plsc.store_scatter(spos_v, [old + cnt - 1], iota + gg * _L)
            plsc.addupdate_scatter(offs_v, [j], cnt, mask=last)
        return carry

    lax.fori_loop(0, _NG // 4, _place, 0)

    lo = (wid * NCOL) >> 5
    hi = ((wid + 1) * NCOL) >> 5
    span = hi - lo

    def _compact(g, ucnt):
        cid = lo + g * _L + iota
        h = hist_v[pl.ds(lo + g * _L, _L)]
        m = (h > 0) & (cid < hi)
        plsc.store_compressed(used_v.at[pl.ds(ucnt, _L)], cid, mask=m)
        return ucnt + plsc.all_reduce_population_count(m)[0]

    ucnt = lax.fori_loop(0, (span + _L - 1) >> 4, _compact, 0)

    def _zrow(i, carry):
        for q in range(DIM // _L):
            rows_v[i, pl.ds(DIM + q * _L, _L)] = zvf
        return carry

    lax.fori_loop(0, _FB, _zrow, 0)

    cvec0 = used_v[pl.ds(0, _L)]
    for k in range(_K):
        @pl.when(k < ucnt)
        def _(k=k):
            coloff = pl.multiple_of(cvec0[k] * 128, 128)
            pltpu.make_async_copy(
                tt_hbm.at[:, pl.ds(coloff, 128)], cols[k], gsems[k]
            ).start()

    def _sweep(o, rc):
        cvec = used_v[pl.ds(o * _K, 2 * _K)]
        for k in range(_K):
            uu = o * _K + k
            live = uu < ucnt

            @pl.when(live)
            def _(k=k):
                pltpu.make_async_copy(
                    tt_hbm.at[:, pl.ds(0, 128)], cols[k], gsems[k]
                ).wait()

            c = jnp.where(live, cvec[k], 0)
            n_c = jnp.where(live, hist_v[pl.ds(c, _L)][0], 0)
            s_c = offs_v[pl.ds(c, _L)][0] - n_c

            def _chunk(t, rc, k=k, n_c=n_c, s_c=s_c):
                rem = n_c - t * _L
                pv = spos_v[pl.ds(s_c + t * _L, _L)]
                lanes = plsc.load_gather(ids_v, [pv], mask=iota < rem) & 127
                for k2 in range(_L):
                    row = (rc + k2) & (_FB - 1)

                    @pl.when(k2 < rem)
                    def _(k2=k2, row=row, pv=pv, lanes=lanes):
                        lanev = jnp.full((_L,), lanes[k2], dtype=jnp.int32)
                        for q in range(DIM // _L):
                            v = plsc.load_gather(cols[k], [iotas[q], lanev])
                            rows_v[row, pl.ds(q * _L, _L)] = v
                        plsc.store_scatter(
                            rpos_v,
                            [jnp.full((_L,), row, dtype=jnp.int32)],
                            jnp.full((_L,), pv[k2], dtype=jnp.int32),
                            mask=m0,
                        )

                    @pl.when((k2 < rem) & (row == _FB - 1))
                    def _():
                        pltpu.sync_copy(rows_v, out_hbm.at[rpos_v])

                return rc + jnp.clip(rem, 0, _L)

            rc = lax.fori_loop(0, (n_c + _L - 1) >> 4, _chunk, rc)

            nu = uu + _K

            @pl.when(nu < ucnt)
            def _(k=k):
                coloff = pl.multiple_of(cvec[k + _K] * 128, 128)
                pltpu.make_async_copy(
                    tt_hbm.at[:, pl.ds(coloff, 128)], cols[k], gsems[k]
                ).start()

        return rc

    rc = lax.fori_loop(0, (ucnt + _K - 1) >> 3, _sweep, 0)

    tail = rc & (_FB - 1)

    @pl.when(tail > 0)
    def _():
        lastp = plsc.load_gather(
            rpos_v, [jnp.full((_L,), tail - 1, dtype=jnp.int32)]
        )

        def _pad(t, carry):
            i = tail + t
            for q in range(2 * DIM // _L):
                rows_v[i, pl.ds(q * _L, _L)] = rows_v[tail - 1, pl.ds(q * _L, _L)]
            return carry

        lax.fori_loop(0, _FB - tail, _pad, 0)

        for t in range(_FB // _L):
            idxv = tail + t * _L + iota
            plsc.store_scatter(rpos_v, [idxv], lastp, mask=idxv < _FB)
        pltpu.sync_copy(rows_v, out_hbm.at[rpos_v])


def kernel(entity_ids, entity_table):
    ids = entity_ids.astype(jnp.int32)
    return _gather_concat(ids, entity_table.T)


# speedup vs baseline: 1.2276x; 1.2231x over previous
import functools

import jax
import jax.numpy as jnp
from jax import lax
from jax.experimental import pallas as pl
from jax.experimental.pallas import tpu as pltpu
from jax.experimental.pallas import tpu_sc as plsc

BATCH = 16384
DIM = 64

_INFO = plsc.get_sparse_core_info()
_NC = _INFO.num_cores
_NS = _INFO.num_subcores
_NW = _NC * _NS
_BPW = BATCH // _NW
_L = 16
_K = 8
_BLK = 128
_NBLK = _BPW // _BLK
_GRP = _BLK // _L

_mesh = plsc.VectorSubcoreMesh(core_axis_name="c", subcore_axis_name="s")


@functools.partial(
    pl.kernel,
    mesh=_mesh,
    out_type=jax.ShapeDtypeStruct((BATCH, 2 * DIM), jnp.float32),
    scratch_types=[
        pltpu.VMEM((_BPW + 2 * _L,), jnp.int32),
        [pltpu.VMEM((DIM, 128), jnp.float32) for _ in range(_K)],
        [pltpu.VMEM((_BLK, 2 * DIM), jnp.float32) for _ in range(2)],
        [pltpu.SemaphoreType.DMA for _ in range(_K)],
        [pltpu.SemaphoreType.DMA for _ in range(2)],
    ],
    compiler_params=pltpu.CompilerParams(needs_layout_passes=False),
)
def _gather_concat(ids_hbm, tt_hbm, out_hbm, ids_v, cols, asms, gsems, osems):
    wid = lax.axis_index("s") * _NC + lax.axis_index("c")
    base = wid * _BPW

    pltpu.sync_copy(ids_hbm.at[pl.ds(base, _BPW)], ids_v.at[pl.ds(0, _BPW)])

    def _fetch(slot, eid):
        coloff = pl.multiple_of((eid >> 7) * 128, 128)
        pltpu.make_async_copy(
            tt_hbm.at[:, pl.ds(coloff, 128)], cols[slot], gsems[slot]
        ).start()

    vec0 = ids_v[pl.ds(0, _L)]
    for k in range(_K):
        _fetch(k, vec0[k])

    zvec = jnp.zeros((_L,), jnp.float32)

    def _zrow(i, carry):
        for b in range(2):
            for q in range(DIM // _L):
                asms[b][i, pl.ds(DIM + q * _L, _L)] = zvec
        return carry

    lax.fori_loop(0, _BLK, _zrow, 0)

    iotas = [lax.iota(jnp.int32, _L) + q * _L for q in range(DIM // _L)]

    for blk in range(_NBLK):
        b = blk % 2
        asm = asms[b]
        if blk >= 2:
            pltpu.make_async_copy(
                asm, out_hbm.at[pl.ds(base + (blk - 2) * _BLK, _BLK)], osems[b]
            ).wait()

        def _group(s, carry, blk=blk, asm=asm):
            g0 = blk * _BLK + s * _L
            vec_c = ids_v[pl.ds(g0, _L)]
            vec_n = ids_v[pl.ds(g0 + _L, _L)]
            for k in range(_L):
                slot = k % _K
                pltpu.make_async_copy(
                    tt_hbm.at[:, pl.ds(0, 128)], cols[slot], gsems[slot]
                ).wait()
                lane = vec_c[k] & 127
                lanev = jnp.full((_L,), lane, dtype=jnp.int32)
                row = s * _L + k
                for q in range(DIM // _L):
                    v = plsc.load_gather(cols[slot], [iotas[q], lanev])
                    asm[row, pl.ds(q * _L, _L)] = v
                nid = vec_c[k + _K] if k < _L - _K else vec_n[k + _K - _L]
                if blk < _NBLK - 1:
                    _fetch(slot, nid)
                else:
                    nxt = g0 + k + _K

                    @pl.when(nxt < _BPW)
                    def _():
                        _fetch(slot, nid)
            return carry

        lax.fori_loop(0, _GRP, _group, 0)
        pltpu.make_async_copy(
            asm, out_hbm.at[pl.ds(base + blk * _BLK, _BLK)], osems[b]
        ).start()

    for blk in (_NBLK - 2, _NBLK - 1):
        b = blk % 2
        pltpu.make_async_copy(
            asms[b], out_hbm.at[pl.ds(base + blk * _BLK, _BLK)], osems[b]
        ).wait()


def kernel(entity_ids, entity_table):
    ids = entity_ids.astype(jnp.int32)
    return _gather_concat(ids, entity_table.T)
